# baseline (speedup 1.0000x reference)
import functools

import jax
import jax.numpy as jnp
from jax import lax
from jax.experimental import pallas as pl
from jax.experimental.pallas import tpu as pltpu
from jax.experimental.pallas import tpu_sc as plsc

N_AUTHORS = 5000
N_PAPERS = 5000
N = N_AUTHORS + N_PAPERS
E = 320000
D = 128
L = 3

NC = 2
NS = 16
NW = NC * NS
CHUNK = 64
NCHUNK = 168
EPAD = NW * NCHUNK * CHUNK
SUP = 24
NSUP = NCHUNK // SUP
NBUF = 3

OBUF_ROWS = 40
NRCHUNK = N // OBUF_ROWS
RITER = -(-NRCHUNK // NS)


def _spmm_body(x_hbm, col_hbm, row_hbm, val_hbm, out_hbm,
               colv, rowv, valv, rows0, rows1, rows2, acc,
               gs0, gs1, gs2, ss, is0, is1):
    cid = lax.axis_index("c")
    sid = lax.axis_index("s")
    wid = cid * NS + sid
    rows = (rows0, rows1, rows2)
    gsem = (gs0, gs1, gs2)
    isem = (is0, is1)
    obuf = rows0.at[pl.ds(0, OBUF_ROWS)]

    zv = jnp.zeros((16,), jnp.float32)

    def zero_row(i, carry):
        for db in range(D // 16):
            rows0[i, pl.ds(db * 16, 16)] = zv
        return carry

    lax.fori_loop(0, OBUF_ROWS, zero_row, 0)
    for kk in range(RITER):
        rc = sid + kk * NS

        @pl.when(rc < NRCHUNK)
        def _():
            pltpu.sync_copy(obuf, acc.at[pl.ds(rc * OBUF_ROWS, OBUF_ROWS)])
    plsc.subcore_barrier()

    c0 = wid * NCHUNK

    def start_idx(s):
        sb = s % 2
        src = pl.ds(c0 + s * SUP, SUP)
        pltpu.async_copy(col_hbm.at[src], colv.at[sb], isem[sb])
        pltpu.async_copy(row_hbm.at[src], rowv.at[sb], isem[sb])
        pltpu.async_copy(val_hbm.at[src], valv.at[sb], isem[sb])

    def wait_idx(s):
        sb = s % 2
        src = pl.ds(c0 + s * SUP, SUP)
        pltpu.make_async_copy(col_hbm.at[src], colv.at[sb], isem[sb]).wait()
        pltpu.make_async_copy(row_hbm.at[src], rowv.at[sb], isem[sb]).wait()
        pltpu.make_async_copy(val_hbm.at[src], valv.at[sb], isem[sb]).wait()

    def start_gather(sb, jj, b):
        pltpu.async_copy(x_hbm.at[colv.at[sb, jj]], rows[b], gsem[b])

    def wait_gather(sb, jj, b):
        pltpu.make_async_copy(x_hbm.at[colv.at[sb, jj]], rows[b],
                              gsem[b]).wait()

    def scatter(sb, jj, b):
        pltpu.async_copy(rows[b], acc.at[rowv.at[sb, jj]], ss, add=True)
        pltpu.make_async_copy(rows[b], acc.at[rowv.at[sb, jj]], ss).wait()

    def scale(sb, jj, b):
        def group_body(g, c2):
            vals16 = valv[sb, jj, pl.ds(g * 16, 16)]
            for t in range(16):
                e = g * 16 + t
                v = vals16[t]
                for db in range(D // 16):
                    sl = pl.ds(db * 16, 16)
                    rows[b][e, sl] = rows[b][e, sl] * v
            return c2

        lax.fori_loop(0, CHUNK // 16, group_body, 0)

    start_idx(0)

    for s in range(NSUP):
        sb = s % 2
        wait_idx(s)
        if s + 1 < NSUP:
            start_idx(s + 1)
        for l in range(NBUF - 1):
            start_gather(sb, l, l)

        def ring_body(p, c2, sb=sb):
            for b in range(NBUF):
                jj = p * NBUF + b
                wait_gather(sb, jj, b)
                scale(sb, jj, b)
                scatter(sb, jj, b)
                jn = jj + NBUF - 1

                @pl.when(jn < SUP)
                def _():
                    start_gather(sb, jn, (b + NBUF - 1) % NBUF)
            return c2

        lax.fori_loop(0, SUP // NBUF, ring_body, 0)

    plsc.subcore_barrier()

    for kk in range(RITER):
        rc = sid + kk * NS

        @pl.when(rc < NRCHUNK)
        def _():
            r0 = rc * OBUF_ROWS
            pltpu.sync_copy(acc.at[pl.ds(r0, OBUF_ROWS)], obuf)
            pltpu.sync_copy(obuf, out_hbm.at[cid, pl.ds(r0, OBUF_ROWS)])


_spmm_sc = functools.partial(
    pl.kernel,
    mesh=plsc.VectorSubcoreMesh(core_axis_name="c", subcore_axis_name="s"),
    out_type=jax.ShapeDtypeStruct((NC, N, D), jnp.float32),
    scratch_types=[
        pltpu.VMEM((2, SUP, CHUNK), jnp.int32),
        pltpu.VMEM((2, SUP, CHUNK), jnp.int32),
        pltpu.VMEM((2, SUP, CHUNK), jnp.float32),
        pltpu.VMEM((CHUNK, D), jnp.float32),
        pltpu.VMEM((CHUNK, D), jnp.float32),
        pltpu.VMEM((CHUNK, D), jnp.float32),
        pltpu.VMEM_SHARED((N, D), jnp.float32),
        pltpu.SemaphoreType.DMA,
        pltpu.SemaphoreType.DMA,
        pltpu.SemaphoreType.DMA,
        pltpu.SemaphoreType.DMA,
        pltpu.SemaphoreType.DMA,
        pltpu.SemaphoreType.DMA,
    ],
)(_spmm_body)


BLK = 1000


def _dense_body(part_ref, ego_ref, w1_ref, b1_ref, w2_ref, b2_ref,
                egon_ref, norm_ref):
    side = part_ref[0] + part_ref[1]
    ego = ego_ref[...]
    s1 = lax.dot_general(side, w1_ref[...], (((1,), (1,)), ((), ())),
                         preferred_element_type=jnp.float32) + b1_ref[...]
    s1 = jnp.where(s1 >= 0, s1, 0.01 * s1)
    s2 = lax.dot_general(ego * side, w2_ref[...], (((1,), (1,)), ((), ())),
                         preferred_element_type=jnp.float32) + b2_ref[...]
    s2 = jnp.where(s2 >= 0, s2, 0.01 * s2)
    e = s1 + s2
    egon_ref[...] = e
    nrm = jnp.sqrt(jnp.sum(e * e, axis=1, keepdims=True))
    norm_ref[...] = e / jnp.maximum(nrm, 1e-12)


_dense_tc = pl.pallas_call(
    _dense_body,
    grid=(N // BLK,),
    in_specs=[
        pl.BlockSpec((NC, BLK, D), lambda i: (0, i, 0)),
        pl.BlockSpec((BLK, D), lambda i: (i, 0)),
        pl.BlockSpec((D, D), lambda i: (0, 0)),
        pl.BlockSpec((1, D), lambda i: (0, 0)),
        pl.BlockSpec((D, D), lambda i: (0, 0)),
        pl.BlockSpec((1, D), lambda i: (0, 0)),
    ],
    out_specs=[
        pl.BlockSpec((BLK, D), lambda i: (i, 0)),
        pl.BlockSpec((BLK, D), lambda i: (i, 0)),
    ],
    out_shape=[
        jax.ShapeDtypeStruct((N, D), jnp.float32),
        jax.ShapeDtypeStruct((N, D), jnp.float32),
    ],
)


def kernel(author_embedding, paper_embedding, adj_values, W1, b1, W2, b2,
           edge_index):
    ego = jnp.concatenate([author_embedding, paper_embedding], axis=0)
    pad = EPAD - E
    row = jnp.pad(edge_index[0], (0, pad)).reshape(NW * NCHUNK, CHUNK)
    col = jnp.pad(edge_index[1], (0, pad)).reshape(NW * NCHUNK, CHUNK)
    val = jnp.pad(adj_values, (0, pad)).reshape(NW * NCHUNK, CHUNK)
    outs = [ego]
    for k in range(L):
        part = _spmm_sc(ego, col, row, val)
        ego, nrm = _dense_tc(part, ego, W1[k], b1[k].reshape(1, D),
                             W2[k], b2[k].reshape(1, D))
        outs.append(nrm)
    all_emb = jnp.concatenate(outs, axis=1)
    return (all_emb[:N_AUTHORS], all_emb[N_AUTHORS:])


# speedup vs baseline: 6.0952x; 6.0952x over previous
import functools

import jax
import jax.numpy as jnp
from jax import lax
from jax.experimental import pallas as pl
from jax.experimental.pallas import tpu as pltpu
from jax.experimental.pallas import tpu_sc as plsc

N_AUTHORS = 5000
N_PAPERS = 5000
N = N_AUTHORS + N_PAPERS
E = 320000
D = 128
L = 3

NC = 2
NS = 16
NW = NC * NS
EPW = E // NW
CHUNK = 80
NCHUNK = EPW // CHUNK

OBUF_ROWS = 80
NRCHUNK = N // OBUF_ROWS
RITER = -(-NRCHUNK // NS)


def _spmm_body(x_hbm, col_hbm, row_hbm, val_hbm, out_hbm,
               colv, rowv, valv, rows0, rows1, acc,
               gs0, gs1, is0, is1):
    cid = lax.axis_index("c")
    sid = lax.axis_index("s")
    wid = cid * NS + sid
    rows = (rows0, rows1)
    gsem = (gs0, gs1)
    isem = (is0, is1)
    obuf = rows0.at[pl.ds(0, OBUF_ROWS)]

    zv = jnp.zeros((16,), jnp.float32)

    def zero_row(i, carry):
        for db in range(D // 16):
            rows0[i, pl.ds(db * 16, 16)] = zv
        return carry

    lax.fori_loop(0, OBUF_ROWS, zero_row, 0)
    for kk in range(RITER):
        rc = sid + kk * NS

        @pl.when(rc < NRCHUNK)
        def _():
            pltpu.sync_copy(obuf, acc.at[pl.ds(rc * OBUF_ROWS, OBUF_ROWS)])
    plsc.subcore_barrier()

    e0 = wid * EPW

    def start_idx(j, ib):
        src = pl.ds(pl.multiple_of(e0 + j * CHUNK, 8), CHUNK)
        pltpu.async_copy(col_hbm.at[src], colv.at[ib], isem[ib])
        pltpu.async_copy(row_hbm.at[src], rowv.at[ib], isem[ib])
        pltpu.async_copy(val_hbm.at[src], valv.at[ib], isem[ib])

    def wait_idx(j, ib):
        src = pl.ds(pl.multiple_of(e0 + j * CHUNK, 8), CHUNK)
        pltpu.make_async_copy(col_hbm.at[src], colv.at[ib], isem[ib]).wait()
        pltpu.make_async_copy(row_hbm.at[src], rowv.at[ib], isem[ib]).wait()
        pltpu.make_async_copy(val_hbm.at[src], valv.at[ib], isem[ib]).wait()

    def start_gather(ib, b):
        pltpu.async_copy(x_hbm.at[colv.at[ib]], rows[b], gsem[b])

    def wait_gather(ib, b):
        pltpu.make_async_copy(x_hbm.at[colv.at[ib]], rows[b], gsem[b]).wait()

    def scale(ib, b):
        def group_body(g, c2):
            vals16 = valv[ib, pl.ds(g * 16, 16)]
            for t in range(16):
                e = g * 16 + t
                v = vals16[t]
                for db in range(D // 16):
                    sl = pl.ds(db * 16, 16)
                    rows[b][e, sl] = rows[b][e, sl] * v
            return c2

        lax.fori_loop(0, CHUNK // 16, group_body, 0)

    def scatter(ib, b):
        pltpu.sync_copy(rows[b], acc.at[rowv.at[ib]], add=True)

    start_idx(0, 0)
    wait_idx(0, 0)
    start_gather(0, 0)
    start_idx(1, 1)

    def chunk_body(p, carry):
        for b in range(2):
            j = 2 * p + b
            nb = 1 - b

            @pl.when(j < NCHUNK)
            def _():
                @pl.when(j + 1 < NCHUNK)
                def _():
                    wait_idx(j + 1, nb)
                    start_gather(nb, nb)

                wait_gather(b, b)
                scale(b, b)
                scatter(b, b)

                @pl.when(j + 2 < NCHUNK)
                def _():
                    start_idx(j + 2, b)
        return carry

    lax.fori_loop(0, (NCHUNK + 1) // 2, chunk_body, 0)
    plsc.subcore_barrier()

    for kk in range(RITER):
        rc = sid + kk * NS

        @pl.when(rc < NRCHUNK)
        def _():
            r0 = rc * OBUF_ROWS
            pltpu.sync_copy(acc.at[pl.ds(r0, OBUF_ROWS)], obuf)
            pltpu.sync_copy(obuf, out_hbm.at[cid, pl.ds(r0, OBUF_ROWS)])


_spmm_sc = functools.partial(
    pl.kernel,
    mesh=plsc.VectorSubcoreMesh(core_axis_name="c", subcore_axis_name="s"),
    out_type=jax.ShapeDtypeStruct((NC, N, D), jnp.float32),
    scratch_types=[
        pltpu.VMEM((2, CHUNK), jnp.int32),
        pltpu.VMEM((2, CHUNK), jnp.int32),
        pltpu.VMEM((2, CHUNK), jnp.float32),
        pltpu.VMEM((CHUNK, D), jnp.float32),
        pltpu.VMEM((CHUNK, D), jnp.float32),
        pltpu.VMEM_SHARED((N, D), jnp.float32),
        pltpu.SemaphoreType.DMA,
        pltpu.SemaphoreType.DMA,
        pltpu.SemaphoreType.DMA,
        pltpu.SemaphoreType.DMA,
    ],
)(_spmm_body)


BLK = 1000


def _dense_body(part_ref, ego_ref, w1_ref, b1_ref, w2_ref, b2_ref,
                egon_ref, norm_ref):
    side = part_ref[0] + part_ref[1]
    ego = ego_ref[...]
    s1 = lax.dot_general(side, w1_ref[...], (((1,), (1,)), ((), ())),
                         preferred_element_type=jnp.float32) + b1_ref[...]
    s1 = jnp.where(s1 >= 0, s1, 0.01 * s1)
    s2 = lax.dot_general(ego * side, w2_ref[...], (((1,), (1,)), ((), ())),
                         preferred_element_type=jnp.float32) + b2_ref[...]
    s2 = jnp.where(s2 >= 0, s2, 0.01 * s2)
    e = s1 + s2
    egon_ref[...] = e
    nrm = jnp.sqrt(jnp.sum(e * e, axis=1, keepdims=True))
    norm_ref[...] = e / jnp.maximum(nrm, 1e-12)


_dense_tc = pl.pallas_call(
    _dense_body,
    grid=(N // BLK,),
    in_specs=[
        pl.BlockSpec((NC, BLK, D), lambda i: (0, i, 0)),
        pl.BlockSpec((BLK, D), lambda i: (i, 0)),
        pl.BlockSpec((D, D), lambda i: (0, 0)),
        pl.BlockSpec((1, D), lambda i: (0, 0)),
        pl.BlockSpec((D, D), lambda i: (0, 0)),
        pl.BlockSpec((1, D), lambda i: (0, 0)),
    ],
    out_specs=[
        pl.BlockSpec((BLK, D), lambda i: (i, 0)),
        pl.BlockSpec((BLK, D), lambda i: (i, 0)),
    ],
    out_shape=[
        jax.ShapeDtypeStruct((N, D), jnp.float32),
        jax.ShapeDtypeStruct((N, D), jnp.float32),
    ],
)


def kernel(author_embedding, paper_embedding, adj_values, W1, b1, W2, b2,
           edge_index):
    ego = jnp.concatenate([author_embedding, paper_embedding], axis=0)
    row = edge_index[0]
    col = edge_index[1]
    outs = [ego]
    for k in range(L):
        part = _spmm_sc(ego, col, row, adj_values)
        ego, nrm = _dense_tc(part, ego, W1[k], b1[k].reshape(1, D),
                             W2[k], b2[k].reshape(1, D))
        outs.append(nrm)
    all_emb = jnp.concatenate(outs, axis=1)
    return (all_emb[:N_AUTHORS], all_emb[N_AUTHORS:])
